# baseline (speedup 1.0000x reference)
import jax
import jax.numpy as jnp
from jax.experimental import pallas as pl

TOKENS = 16384
HIDDEN = 4096
EXPERTS = 64

BM = 1024
BN = 1024


def _router_body(x_ref, w1_ref, b1_ref, w2_ref, b2_ref, o_ref):
    n = pl.program_id(1)
    h = jnp.dot(x_ref[...], w1_ref[...], preferred_element_type=jnp.float32)
    h = jnp.maximum(h + b1_ref[...], 0.0)
    p = jnp.dot(h.astype(jnp.bfloat16), w2_ref[...],
                preferred_element_type=jnp.float32)

    @pl.when(n == 0)
    def _init():
        o_ref[...] = p + b2_ref[...]

    @pl.when(n != 0)
    def _acc():
        o_ref[...] += p


def kernel(x, W1, b1, W2, b2):
    xb = x.astype(jnp.bfloat16)
    w1t = W1.T.astype(jnp.bfloat16)
    w2t = W2.T.astype(jnp.bfloat16)
    b1r = b1.reshape(1, HIDDEN)
    b2r = b2.reshape(1, EXPERTS)

    grid = (TOKENS // BM, HIDDEN // BN)
    return pl.pallas_call(
        _router_body,
        grid=grid,
        in_specs=[
            pl.BlockSpec((BM, HIDDEN), lambda m, n: (m, 0)),
            pl.BlockSpec((HIDDEN, BN), lambda m, n: (0, n)),
            pl.BlockSpec((1, BN), lambda m, n: (0, n)),
            pl.BlockSpec((BN, EXPERTS), lambda m, n: (n, 0)),
            pl.BlockSpec((1, EXPERTS), lambda m, n: (0, 0)),
        ],
        out_specs=pl.BlockSpec((BM, EXPERTS), lambda m, n: (m, 0)),
        out_shape=jax.ShapeDtypeStruct((TOKENS, EXPERTS), jnp.float32),
    )(xb, w1t, b1r, w2t, b2r)


# speedup vs baseline: 1.0340x; 1.0340x over previous
import jax
import jax.numpy as jnp
from jax.experimental import pallas as pl

TOKENS = 16384
HIDDEN = 4096
EXPERTS = 64

BM = 1024
BN = 512


def _router_body(x_ref, w1_ref, b1_ref, w2_ref, b2_ref, o_ref):
    n = pl.program_id(1)
    h = jnp.dot(x_ref[...], w1_ref[...], preferred_element_type=jnp.float32)
    h = jnp.maximum(h + b1_ref[...], 0.0)
    p = jnp.dot(h, w2_ref[...], preferred_element_type=jnp.float32)

    @pl.when(n == 0)
    def _init():
        o_ref[...] = p + b2_ref[...]

    @pl.when(n != 0)
    def _acc():
        o_ref[...] += p


def kernel(x, W1, b1, W2, b2):
    w1t = W1.T
    w2t = W2.T
    b1r = b1.reshape(1, HIDDEN)
    b2r = b2.reshape(1, EXPERTS)

    grid = (TOKENS // BM, HIDDEN // BN)
    return pl.pallas_call(
        _router_body,
        grid=grid,
        in_specs=[
            pl.BlockSpec((BM, HIDDEN), lambda m, n: (m, 0)),
            pl.BlockSpec((HIDDEN, BN), lambda m, n: (0, n)),
            pl.BlockSpec((1, BN), lambda m, n: (0, n)),
            pl.BlockSpec((BN, EXPERTS), lambda m, n: (n, 0)),
            pl.BlockSpec((1, EXPERTS), lambda m, n: (0, 0)),
        ],
        out_specs=pl.BlockSpec((BM, EXPERTS), lambda m, n: (m, 0)),
        out_shape=jax.ShapeDtypeStruct((TOKENS, EXPERTS), jnp.float32),
    )(x, w1t, b1r, w2t, b2r)


# speedup vs baseline: 1.0593x; 1.0245x over previous
import jax
import jax.numpy as jnp
from jax.experimental import pallas as pl
from jax.experimental.pallas import tpu as pltpu

TOKENS = 16384
HIDDEN = 4096
EXPERTS = 64

BM = 1024
BN = 512


def _router_body(x_ref, w1_ref, b1_ref, w2_ref, b2_ref, o_ref, xb_ref):
    n = pl.program_id(1)

    @pl.when(n == 0)
    def _cast():
        xb_ref[...] = x_ref[...].astype(jnp.bfloat16)

    h = jnp.dot(xb_ref[...], w1_ref[...], preferred_element_type=jnp.float32)
    h = jnp.maximum(h + b1_ref[...], 0.0)
    p = jnp.dot(h, w2_ref[...], preferred_element_type=jnp.float32)

    @pl.when(n == 0)
    def _init():
        o_ref[...] = p + b2_ref[...]

    @pl.when(n != 0)
    def _acc():
        o_ref[...] += p


def kernel(x, W1, b1, W2, b2):
    w1t = W1.T.astype(jnp.bfloat16)
    w2t = W2.T
    b1r = b1.reshape(1, HIDDEN)
    b2r = b2.reshape(1, EXPERTS)

    grid = (TOKENS // BM, HIDDEN // BN)
    return pl.pallas_call(
        _router_body,
        grid=grid,
        in_specs=[
            pl.BlockSpec((BM, HIDDEN), lambda m, n: (m, 0)),
            pl.BlockSpec((HIDDEN, BN), lambda m, n: (0, n)),
            pl.BlockSpec((1, BN), lambda m, n: (0, n)),
            pl.BlockSpec((BN, EXPERTS), lambda m, n: (n, 0)),
            pl.BlockSpec((1, EXPERTS), lambda m, n: (0, 0)),
        ],
        out_specs=pl.BlockSpec((BM, EXPERTS), lambda m, n: (m, 0)),
        out_shape=jax.ShapeDtypeStruct((TOKENS, EXPERTS), jnp.float32),
        scratch_shapes=[pltpu.VMEM((BM, HIDDEN), jnp.bfloat16)],
    )(x, w1t, b1r, w2t, b2r)


# speedup vs baseline: 1.1325x; 1.0691x over previous
import jax
import jax.numpy as jnp
from jax.experimental import pallas as pl
from jax.experimental.pallas import tpu as pltpu

TOKENS = 16384
HIDDEN = 4096
EXPERTS = 64

BM = 512
BN = 512
NCHUNK = HIDDEN // BN

_DN = (((1,), (1,)), ((), ()))


def _router_body(x_ref, w1_ref, b1_ref, w2_ref, b2_ref, o_ref, xb_ref):
    xb_ref[...] = x_ref[...].astype(jnp.bfloat16)
    xb = xb_ref[...]
    acc = jnp.broadcast_to(b2_ref[...], (BM, EXPERTS))
    for j in range(NCHUNK):
        w1c = w1_ref[pl.ds(j * BN, BN), :]
        h = jax.lax.dot_general(xb, w1c, _DN,
                                preferred_element_type=jnp.float32)
        h = jnp.maximum(h + b1_ref[:, pl.ds(j * BN, BN)], 0.0)
        w2c = w2_ref[:, pl.ds(j * BN, BN)]
        acc = acc + jax.lax.dot_general(h, w2c, _DN,
                                        preferred_element_type=jnp.float32)
    o_ref[...] = acc


def kernel(x, W1, b1, W2, b2):
    w1b = W1.astype(jnp.bfloat16)
    b1r = b1.reshape(1, HIDDEN)
    b2r = b2.reshape(1, EXPERTS)

    grid = (TOKENS // BM,)
    return pl.pallas_call(
        _router_body,
        grid=grid,
        in_specs=[
            pl.BlockSpec((BM, HIDDEN), lambda m: (m, 0)),
            pl.BlockSpec((HIDDEN, HIDDEN), lambda m: (0, 0)),
            pl.BlockSpec((1, HIDDEN), lambda m: (0, 0)),
            pl.BlockSpec((EXPERTS, HIDDEN), lambda m: (0, 0)),
            pl.BlockSpec((1, EXPERTS), lambda m: (0, 0)),
        ],
        out_specs=pl.BlockSpec((BM, EXPERTS), lambda m: (m, 0)),
        out_shape=jax.ShapeDtypeStruct((TOKENS, EXPERTS), jnp.float32),
        scratch_shapes=[pltpu.VMEM((BM, HIDDEN), jnp.bfloat16)],
    )(x, w1b, b1r, W2, b2r)


# speedup vs baseline: 1.2335x; 1.0892x over previous
import jax
import jax.numpy as jnp
from jax.experimental import pallas as pl
from jax.experimental.pallas import tpu as pltpu

TOKENS = 16384
HIDDEN = 4096
EXPERTS = 64

BM = 512
BN = 4096
NCHUNK = HIDDEN // BN

_DN = (((1,), (1,)), ((), ()))


_KS = 8


def _router_body(x_ref, w1_ref, b1_ref, w2_ref, b2_ref, o_ref, xb_ref):
    acc = jnp.broadcast_to(b2_ref[...], (BM, EXPERTS))

    ksz = HIDDEN // _KS
    h = None
    for k in range(_KS):
        xk = x_ref[:, pl.ds(k * ksz, ksz)].astype(jnp.bfloat16)
        xb_ref[:, pl.ds(k * ksz, ksz)] = xk
        hk = jax.lax.dot_general(
            xk, w1_ref[pl.ds(0, BN), pl.ds(k * ksz, ksz)], _DN,
            preferred_element_type=jnp.float32)
        h = hk if h is None else h + hk
    h = jnp.maximum(h + b1_ref[:, pl.ds(0, BN)], 0.0)
    acc = acc + jax.lax.dot_general(h, w2_ref[:, pl.ds(0, BN)], _DN,
                                    preferred_element_type=jnp.float32)

    xb = xb_ref[...]
    for j in range(1, NCHUNK):
        w1c = w1_ref[pl.ds(j * BN, BN), :]
        h = jax.lax.dot_general(xb, w1c, _DN,
                                preferred_element_type=jnp.float32)
        h = jnp.maximum(h + b1_ref[:, pl.ds(j * BN, BN)], 0.0)
        w2c = w2_ref[:, pl.ds(j * BN, BN)]
        acc = acc + jax.lax.dot_general(h, w2c, _DN,
                                        preferred_element_type=jnp.float32)
    o_ref[...] = acc


def kernel(x, W1, b1, W2, b2):
    w1b = W1.astype(jnp.bfloat16)
    b1r = b1.reshape(1, HIDDEN)
    b2r = b2.reshape(1, EXPERTS)

    grid = (TOKENS // BM,)
    return pl.pallas_call(
        _router_body,
        grid=grid,
        in_specs=[
            pl.BlockSpec((BM, HIDDEN), lambda m: (m, 0)),
            pl.BlockSpec((HIDDEN, HIDDEN), lambda m: (0, 0)),
            pl.BlockSpec((1, HIDDEN), lambda m: (0, 0)),
            pl.BlockSpec((EXPERTS, HIDDEN), lambda m: (0, 0)),
            pl.BlockSpec((1, EXPERTS), lambda m: (0, 0)),
        ],
        out_specs=pl.BlockSpec((BM, EXPERTS), lambda m: (m, 0)),
        out_shape=jax.ShapeDtypeStruct((TOKENS, EXPERTS), jnp.float32),
        scratch_shapes=[pltpu.VMEM((BM, HIDDEN), jnp.bfloat16)],
    )(x, w1b, b1r, W2, b2r)
